# Initial kernel scaffold; baseline (speedup 1.0000x reference)
import jax
import jax.numpy as jnp
from jax.experimental import pallas as pl


def kernel(x, edge_index, batch, W0, b0, Wq, bq, Wk, bk, Wv, bv, Ws, bs, filt, Wsp, bsp, Wg, bg):
    raise NotImplementedError("write your pallas kernel here")


# speedup vs baseline: 3.0223x; 3.0223x over previous
import functools

import jax
import jax.numpy as jnp
from jax import lax
from jax.experimental import pallas as pl
from jax.experimental.pallas import tpu as pltpu
from jax.experimental.pallas import tpu_sc as plsc

N = 10000
E = 160000
D = 256
HEADS = 4
HD = HEADS * D
G = 64
NL = 3

NC, NS, LANES = 2, 16, 16
NW = NC * NS

NP = 10240
EP = 163840
EW = EP // NW
EB = 16
ROWS_PER_SUB = NP // NS
NCHUNK = 8
CW = 128


BM = 512


def _mm_body(x_ref, w_ref, b_ref, o_ref):
  o_ref[...] = jnp.dot(x_ref[...], w_ref[...],
                       preferred_element_type=jnp.float32) + b_ref[...]


def tc_matmul(x, w, b):
  n = x.shape[0]
  return pl.pallas_call(
      _mm_body,
      grid=(n // BM,),
      in_specs=[
          pl.BlockSpec((BM, x.shape[1]), lambda i: (i, 0)),
          pl.BlockSpec(w.shape, lambda i: (0, 0)),
          pl.BlockSpec((1, w.shape[1]), lambda i: (0, 0)),
      ],
      out_specs=pl.BlockSpec((BM, w.shape[1]), lambda i: (i, 0)),
      out_shape=jax.ShapeDtypeStruct((n, w.shape[1]), jnp.float32),
  )(x, w, b.reshape(1, -1))


def _qkvs_body(h_ref, wq_ref, bq_ref, wk_ref, bk_ref, wv_ref, bv_ref,
               ws_ref, bs_ref, q_ref, k_ref, v_ref, s_ref):
  h = h_ref[...]
  q_ref[...] = jnp.dot(h, wq_ref[...], preferred_element_type=jnp.float32) + bq_ref[...]
  k_ref[...] = jnp.dot(h, wk_ref[...], preferred_element_type=jnp.float32) + bk_ref[...]
  v = jnp.dot(h, wv_ref[...], preferred_element_type=jnp.float32) + bv_ref[...]
  for c in range(NCHUNK):
    v_ref[c, :, :] = v[:, c * CW:(c + 1) * CW]
  s_ref[...] = jnp.dot(h, ws_ref[...], preferred_element_type=jnp.float32) + bs_ref[...]


def tc_qkvs(h, wq, bq, wk, bk, wv, bv, ws, bs):
  return pl.pallas_call(
      _qkvs_body,
      grid=(NP // BM,),
      in_specs=[
          pl.BlockSpec((BM, D), lambda i: (i, 0)),
          pl.BlockSpec((D, HD), lambda i: (0, 0)),
          pl.BlockSpec((1, HD), lambda i: (0, 0)),
          pl.BlockSpec((D, HD), lambda i: (0, 0)),
          pl.BlockSpec((1, HD), lambda i: (0, 0)),
          pl.BlockSpec((D, HD), lambda i: (0, 0)),
          pl.BlockSpec((1, HD), lambda i: (0, 0)),
          pl.BlockSpec((D, D), lambda i: (0, 0)),
          pl.BlockSpec((1, D), lambda i: (0, 0)),
      ],
      out_specs=[
          pl.BlockSpec((BM, HD), lambda i: (i, 0)),
          pl.BlockSpec((BM, HD), lambda i: (i, 0)),
          pl.BlockSpec((NCHUNK, BM, CW), lambda i: (0, i, 0)),
          pl.BlockSpec((BM, D), lambda i: (i, 0)),
      ],
      out_shape=[
          jax.ShapeDtypeStruct((NP, HD), jnp.float32),
          jax.ShapeDtypeStruct((NP, HD), jnp.float32),
          jax.ShapeDtypeStruct((NCHUNK, NP, CW), jnp.float32),
          jax.ShapeDtypeStruct((NP, D), jnp.float32),
      ],
  )(h, wq, bq.reshape(1, -1), wk, bk.reshape(1, -1), wv, bv.reshape(1, -1),
    ws, bs.reshape(1, -1))


def _combine_body(m_ref, d_ref, s_ref, o_ref):
  d = d_ref[...]
  for half in range(2):
    acc = None
    for h in range(HEADS):
      inv = 1.0 / (d[h, :] + 1e-16)
      t = m_ref[2 * h + half, :, :] * inv[:, None]
      acc = t if acc is None else acc + t
    o_ref[:, half * CW:(half + 1) * CW] = jnp.maximum(
        acc * (1.0 / HEADS) + s_ref[:, half * CW:(half + 1) * CW], 0.0)


def tc_combine(msg, denom, s):
  return pl.pallas_call(
      _combine_body,
      grid=(NP // BM,),
      in_specs=[
          pl.BlockSpec((NCHUNK, BM, CW), lambda i: (0, i, 0)),
          pl.BlockSpec((HEADS, BM), lambda i: (0, i)),
          pl.BlockSpec((BM, D), lambda i: (i, 0)),
      ],
      out_specs=pl.BlockSpec((BM, D), lambda i: (i, 0)),
      out_shape=jax.ShapeDtypeStruct((NP, D), jnp.float32),
  )(msg, denom, s)


def _pool_body(b_ref, h_ref, o_ref):
  i = pl.program_id(0)

  @pl.when(i == 0)
  def _():
    o_ref[...] = jnp.zeros_like(o_ref)

  b = b_ref[0, 0, :]
  onehot = (b[:, None] == lax.broadcasted_iota(jnp.int32, (1, G), 1)
            ).astype(jnp.float32)
  o_ref[...] += lax.dot_general(onehot, h_ref[...], (((0,), (0,)), ((), ())),
                                preferred_element_type=jnp.float32)


def tc_pool(h, batch3):
  return pl.pallas_call(
      _pool_body,
      grid=(NP // BM,),
      in_specs=[
          pl.BlockSpec((1, 1, BM), lambda i: (i, 0, 0)),
          pl.BlockSpec((BM, D), lambda i: (i, 0)),
      ],
      out_specs=pl.BlockSpec((G, D), lambda i: (0, 0)),
      out_shape=jax.ShapeDtypeStruct((G, D), jnp.float32),
  )(batch3, h)


def _head_body(sp_ref, filt_ref, wsp_ref, bsp_ref, wg1_ref, wg2_ref, bg_ref,
               o_ref):
  spatial = sp_ref[...]
  spectral = jnp.tanh(
      jnp.dot(spatial * filt_ref[...], wsp_ref[...],
              preferred_element_type=jnp.float32) + bsp_ref[...])
  logit = (jnp.sum(spatial * wg1_ref[...], axis=-1, keepdims=True)
           + jnp.sum(spectral * wg2_ref[...], axis=-1, keepdims=True)
           + bg_ref[0, 0])
  g = jax.nn.sigmoid(logit)
  o_ref[...] = g * spatial + (1.0 - g) * spectral


def tc_head(spatial, filt, wsp, bsp, wg, bg):
  wg1 = wg[:D, 0].reshape(1, D)
  wg2 = wg[D:, 0].reshape(1, D)
  return pl.pallas_call(
      _head_body,
      in_specs=[pl.BlockSpec((G, D), lambda: (0, 0)),
                pl.BlockSpec((1, D), lambda: (0, 0)),
                pl.BlockSpec((D, D), lambda: (0, 0)),
                pl.BlockSpec((1, D), lambda: (0, 0)),
                pl.BlockSpec((1, D), lambda: (0, 0)),
                pl.BlockSpec((1, D), lambda: (0, 0)),
                pl.BlockSpec((1, 1), lambda: (0, 0))],
      out_specs=pl.BlockSpec((G, D), lambda: (0, 0)),
      out_shape=jax.ShapeDtypeStruct((G, D), jnp.float32),
  )(spatial, filt.reshape(1, D), wsp, bsp.reshape(1, D), wg1, wg2,
    bg.reshape(1, 1))


_MESH = plsc.VectorSubcoreMesh(core_axis_name="c", subcore_axis_name="s",
                               num_cores=NC, num_subcores=NS)


def _alpha_body(q_hbm, k_hbm, src_hbm, dst_hbm, alpha_out, mx_out,
                idxs_v, idxd_v, qr_v, kr_v, al_v, mv_v, sem_q, sem_k):
  cid = lax.axis_index("c")
  sid = lax.axis_index("s")
  wid = sid * NC + cid
  base = wid * EW
  lane = lax.broadcasted_iota(jnp.int32, (LANES,), 0)

  def batch_body(i, mvec):
    eb = base + i * EB
    pltpu.sync_copy(src_hbm.at[pl.ds(eb, EB)], idxs_v)
    pltpu.sync_copy(dst_hbm.at[pl.ds(eb, EB)], idxd_v)
    cpq = pltpu.async_copy(q_hbm.at[idxd_v], qr_v, sem_q)
    cpk = pltpu.async_copy(k_hbm.at[idxs_v], kr_v, sem_k)
    cpq.wait()
    cpk.wait()

    def edge_body(j, carry):
      outs = []
      for h in range(HEADS):
        acc = jnp.zeros((LANES,), jnp.float32)
        for t in range(D // LANES):
          off = h * D + t * LANES
          acc = acc + qr_v[j, pl.ds(off, LANES)] * kr_v[j, pl.ds(off, LANES)]
        sc = jnp.sum(acc) * (1.0 / 16.0)
        outs.append(jnp.where(lane == j, sc, carry[h]))
      return tuple(outs)

    avecs = lax.fori_loop(0, EB, edge_body,
                          tuple(jnp.zeros((LANES,), jnp.float32)
                                for _ in range(HEADS)))
    for h in range(HEADS):
      al_v[h, pl.ds(i * EB, EB)] = avecs[h]
    m = jnp.maximum(jnp.maximum(avecs[0], avecs[1]),
                    jnp.maximum(avecs[2], avecs[3]))
    return jnp.maximum(mvec, m)

  mvec = lax.fori_loop(0, EW // EB, batch_body,
                       jnp.full((LANES,), -1e30, jnp.float32))
  mv_v[...] = mvec
  for h in range(HEADS):
    pltpu.sync_copy(al_v.at[h], alpha_out.at[h, pl.ds(base, EW)])
  pltpu.sync_copy(mv_v, mx_out.at[wid])


def sc_alpha(q, k, src, dst):
  kfn = pl.kernel(
      _alpha_body,
      out_type=(jax.ShapeDtypeStruct((HEADS, EP), jnp.float32),
                jax.ShapeDtypeStruct((NW, LANES), jnp.float32)),
      mesh=_MESH,
      scratch_types=[
          pltpu.VMEM((EB,), jnp.int32),
          pltpu.VMEM((EB,), jnp.int32),
          pltpu.VMEM((EB, HD), jnp.float32),
          pltpu.VMEM((EB, HD), jnp.float32),
          pltpu.VMEM((HEADS, EW), jnp.float32),
          pltpu.VMEM((LANES,), jnp.float32),
          pltpu.SemaphoreType.DMA,
          pltpu.SemaphoreType.DMA,
      ],
      compiler_params=pltpu.CompilerParams(needs_layout_passes=False),
  )
  return kfn(q, k, src, dst)


EPS = EP // NS


def _scatter_body(v_hbm, src_hbm, dst_hbm, alpha_hbm, mx_hbm, z2_hbm, z1_hbm,
                  msg_out, den_out,
                  idxs_v, idxd_v, vr_v, ea_v, mxa_v, sem_v,
                  msh, dsh0, dsh1):
  cid = lax.axis_index("c")
  sid = lax.axis_index("s")

  pltpu.sync_copy(mx_hbm, mxa_v)
  m16 = jnp.full((LANES,), -1e30, jnp.float32)
  for w in range(NW):
    m16 = jnp.maximum(m16, mxa_v[w, :])
  gmax = jnp.max(m16)

  for ci in range(NCHUNK):
    h = ci // 2
    dsh = dsh0 if (h % 2) == 0 else dsh1

    @pl.when(cid == ci // 4)
    def _(ci=ci, h=h, dsh=dsh):
      sl = pl.ds(sid * ROWS_PER_SUB, ROWS_PER_SUB)
      pltpu.sync_copy(z2_hbm.at[sl], msh.at[sl])
      if ci % 2 == 0:
        pltpu.sync_copy(z1_hbm.at[sl], dsh.at[sl])
      plsc.subcore_barrier()

      def ebody(i, _):
        eb = sid * EPS + i * EB
        pltpu.sync_copy(src_hbm.at[pl.ds(eb, EB)], idxs_v)
        pltpu.sync_copy(dst_hbm.at[pl.ds(eb, EB)], idxd_v)
        pltpu.sync_copy(alpha_hbm.at[h, pl.ds(eb, EB)], ea_v)
        ea_v[...] = jnp.exp(ea_v[...] - gmax)
        pltpu.async_copy(v_hbm.at[ci].at[idxs_v], vr_v, sem_v).wait()

        def sbody(j, _2):
          s16 = plsc.load_gather(ea_v, [jnp.zeros((LANES,), jnp.int32) + j])
          for t in range(CW // LANES):
            vr_v[j, pl.ds(t * LANES, LANES)] = (
                vr_v[j, pl.ds(t * LANES, LANES)] * s16)
          return 0

        lax.fori_loop(0, EB, sbody, 0)
        pltpu.sync_copy(vr_v, msh.at[idxd_v], add=True)
        if ci % 2 == 0:
          pltpu.sync_copy(ea_v, dsh.at[idxd_v], add=True)
        return 0

      lax.fori_loop(0, EPS // EB, ebody, 0)
      plsc.subcore_barrier()
      pltpu.sync_copy(msh.at[pl.ds(sid * ROWS_PER_SUB, ROWS_PER_SUB)],
                      msg_out.at[ci, pl.ds(sid * ROWS_PER_SUB, ROWS_PER_SUB)])
      plsc.subcore_barrier()

  pltpu.sync_copy(dsh0.at[pl.ds(sid * ROWS_PER_SUB, ROWS_PER_SUB)],
                  den_out.at[2 * cid, pl.ds(sid * ROWS_PER_SUB, ROWS_PER_SUB)])
  pltpu.sync_copy(dsh1.at[pl.ds(sid * ROWS_PER_SUB, ROWS_PER_SUB)],
                  den_out.at[2 * cid + 1,
                             pl.ds(sid * ROWS_PER_SUB, ROWS_PER_SUB)])


def sc_scatter(vch, src, dst, alpha4, mx, z2, z1):
  kfn = pl.kernel(
      _scatter_body,
      out_type=(jax.ShapeDtypeStruct((NCHUNK, NP, CW), jnp.float32),
                jax.ShapeDtypeStruct((HEADS, NP), jnp.float32)),
      mesh=_MESH,
      scratch_types=[
          pltpu.VMEM((EB,), jnp.int32),
          pltpu.VMEM((EB,), jnp.int32),
          pltpu.VMEM((EB, CW), jnp.float32),
          pltpu.VMEM((EB,), jnp.float32),
          pltpu.VMEM((NW, LANES), jnp.float32),
          pltpu.SemaphoreType.DMA,
          pltpu.VMEM_SHARED((NP, CW), jnp.float32),
          pltpu.VMEM_SHARED((NP,), jnp.float32),
          pltpu.VMEM_SHARED((NP,), jnp.float32),
      ],
      compiler_params=pltpu.CompilerParams(needs_layout_passes=False),
  )
  return kfn(vch, src, dst, alpha4, mx, z2, z1)


def kernel(x, edge_index, batch, W0, b0, Wq, bq, Wk, bk, Wv, bv, Ws, bs,
           filt, Wsp, bsp, Wg, bg):
  src = jnp.pad(edge_index[0], (0, EP - E), constant_values=NP - 1)
  dst = jnp.pad(edge_index[1], (0, EP - E), constant_values=NP - 1)
  batch3 = jnp.pad(batch, (0, NP - N), constant_values=G).reshape(
      NP // BM, 1, BM)
  xp = jnp.pad(x, ((0, NP - N), (0, 0)))
  z2 = jnp.zeros((NP, CW), jnp.float32)
  z1 = jnp.zeros((NP,), jnp.float32)

  h = tc_matmul(xp, W0, b0)
  for l in range(NL):
    q, k, vch, s = tc_qkvs(h, Wq[l], bq[l], Wk[l], bk[l], Wv[l], bv[l],
                           Ws[l], bs[l])
    alpha4, mx = sc_alpha(q, k, src, dst)
    msg, denom = sc_scatter(vch, src, dst, alpha4, mx, z2, z1)
    h = tc_combine(msg, denom, s)
  spatial = tc_pool(h, batch3)
  return tc_head(spatial, filt, Wsp, bsp, Wg, bg)


# speedup vs baseline: 8.7867x; 2.9073x over previous
import functools

import jax
import jax.numpy as jnp
from jax import lax
from jax.experimental import pallas as pl
from jax.experimental.pallas import tpu as pltpu
from jax.experimental.pallas import tpu_sc as plsc

N = 10000
E = 160000
D = 256
HEADS = 4
HD = HEADS * D
G = 64
NL = 3

NC, NS, LANES = 2, 16, 16
NW = NC * NS

NP = 10240
EP = 163840
EW = EP // NW
EB = 16
ROWS_PER_SUB = NP // NS
NCHUNK = 8
CW = 128


BM = 512


def _mm_body(x_ref, w_ref, b_ref, o_ref):
  o_ref[...] = jnp.dot(x_ref[...], w_ref[...],
                       preferred_element_type=jnp.float32) + b_ref[...]


def tc_matmul(x, w, b):
  n = x.shape[0]
  return pl.pallas_call(
      _mm_body,
      grid=(n // BM,),
      in_specs=[
          pl.BlockSpec((BM, x.shape[1]), lambda i: (i, 0)),
          pl.BlockSpec(w.shape, lambda i: (0, 0)),
          pl.BlockSpec((1, w.shape[1]), lambda i: (0, 0)),
      ],
      out_specs=pl.BlockSpec((BM, w.shape[1]), lambda i: (i, 0)),
      out_shape=jax.ShapeDtypeStruct((n, w.shape[1]), jnp.float32),
  )(x, w, b.reshape(1, -1))


def _qkvs_body(h_ref, wq_ref, bq_ref, wk_ref, bk_ref, wv_ref, bv_ref,
               ws_ref, bs_ref, q_ref, k_ref, v_ref, s_ref):
  h = h_ref[...]
  q_ref[...] = jnp.dot(h, wq_ref[...], preferred_element_type=jnp.float32) + bq_ref[...]
  k_ref[...] = jnp.dot(h, wk_ref[...], preferred_element_type=jnp.float32) + bk_ref[...]
  v = jnp.dot(h, wv_ref[...], preferred_element_type=jnp.float32) + bv_ref[...]
  for c in range(NCHUNK):
    v_ref[c, :, :] = v[:, c * CW:(c + 1) * CW]
  s_ref[...] = jnp.dot(h, ws_ref[...], preferred_element_type=jnp.float32) + bs_ref[...]


def tc_qkvs(h, wq, bq, wk, bk, wv, bv, ws, bs):
  return pl.pallas_call(
      _qkvs_body,
      grid=(NP // BM,),
      in_specs=[
          pl.BlockSpec((BM, D), lambda i: (i, 0)),
          pl.BlockSpec((D, HD), lambda i: (0, 0)),
          pl.BlockSpec((1, HD), lambda i: (0, 0)),
          pl.BlockSpec((D, HD), lambda i: (0, 0)),
          pl.BlockSpec((1, HD), lambda i: (0, 0)),
          pl.BlockSpec((D, HD), lambda i: (0, 0)),
          pl.BlockSpec((1, HD), lambda i: (0, 0)),
          pl.BlockSpec((D, D), lambda i: (0, 0)),
          pl.BlockSpec((1, D), lambda i: (0, 0)),
      ],
      out_specs=[
          pl.BlockSpec((BM, HD), lambda i: (i, 0)),
          pl.BlockSpec((BM, HD), lambda i: (i, 0)),
          pl.BlockSpec((NCHUNK, BM, CW), lambda i: (0, i, 0)),
          pl.BlockSpec((BM, D), lambda i: (i, 0)),
      ],
      out_shape=[
          jax.ShapeDtypeStruct((NP, HD), jnp.float32),
          jax.ShapeDtypeStruct((NP, HD), jnp.float32),
          jax.ShapeDtypeStruct((NCHUNK, NP, CW), jnp.float32),
          jax.ShapeDtypeStruct((NP, D), jnp.float32),
      ],
  )(h, wq, bq.reshape(1, -1), wk, bk.reshape(1, -1), wv, bv.reshape(1, -1),
    ws, bs.reshape(1, -1))


def _combine_body(m_ref, d_ref, s_ref, o_ref):
  d = d_ref[...]
  for half in range(2):
    acc = None
    for h in range(HEADS):
      inv = 1.0 / (d[h, :] + 1e-16)
      t = m_ref[2 * h + half, :, :] * inv[:, None]
      acc = t if acc is None else acc + t
    o_ref[:, half * CW:(half + 1) * CW] = jnp.maximum(
        acc * (1.0 / HEADS) + s_ref[:, half * CW:(half + 1) * CW], 0.0)


def tc_combine(msg, denom, s):
  return pl.pallas_call(
      _combine_body,
      grid=(NP // BM,),
      in_specs=[
          pl.BlockSpec((NCHUNK, BM, CW), lambda i: (0, i, 0)),
          pl.BlockSpec((HEADS, BM), lambda i: (0, i)),
          pl.BlockSpec((BM, D), lambda i: (i, 0)),
      ],
      out_specs=pl.BlockSpec((BM, D), lambda i: (i, 0)),
      out_shape=jax.ShapeDtypeStruct((NP, D), jnp.float32),
  )(msg, denom, s)


def _pool_body(b_ref, h_ref, o_ref):
  i = pl.program_id(0)

  @pl.when(i == 0)
  def _():
    o_ref[...] = jnp.zeros_like(o_ref)

  b = b_ref[0, 0, :]
  onehot = (b[:, None] == lax.broadcasted_iota(jnp.int32, (1, G), 1)
            ).astype(jnp.float32)
  o_ref[...] += lax.dot_general(onehot, h_ref[...], (((0,), (0,)), ((), ())),
                                preferred_element_type=jnp.float32)


def tc_pool(h, batch3):
  return pl.pallas_call(
      _pool_body,
      grid=(NP // BM,),
      in_specs=[
          pl.BlockSpec((1, 1, BM), lambda i: (i, 0, 0)),
          pl.BlockSpec((BM, D), lambda i: (i, 0)),
      ],
      out_specs=pl.BlockSpec((G, D), lambda i: (0, 0)),
      out_shape=jax.ShapeDtypeStruct((G, D), jnp.float32),
  )(batch3, h)


def _head_body(sp_ref, filt_ref, wsp_ref, bsp_ref, wg1_ref, wg2_ref, bg_ref,
               o_ref):
  spatial = sp_ref[...]
  spectral = jnp.tanh(
      jnp.dot(spatial * filt_ref[...], wsp_ref[...],
              preferred_element_type=jnp.float32) + bsp_ref[...])
  logit = (jnp.sum(spatial * wg1_ref[...], axis=-1, keepdims=True)
           + jnp.sum(spectral * wg2_ref[...], axis=-1, keepdims=True)
           + bg_ref[0, 0])
  g = jax.nn.sigmoid(logit)
  o_ref[...] = g * spatial + (1.0 - g) * spectral


def tc_head(spatial, filt, wsp, bsp, wg, bg):
  wg1 = wg[:D, 0].reshape(1, D)
  wg2 = wg[D:, 0].reshape(1, D)
  return pl.pallas_call(
      _head_body,
      in_specs=[pl.BlockSpec((G, D), lambda: (0, 0)),
                pl.BlockSpec((1, D), lambda: (0, 0)),
                pl.BlockSpec((D, D), lambda: (0, 0)),
                pl.BlockSpec((1, D), lambda: (0, 0)),
                pl.BlockSpec((1, D), lambda: (0, 0)),
                pl.BlockSpec((1, D), lambda: (0, 0)),
                pl.BlockSpec((1, 1), lambda: (0, 0))],
      out_specs=pl.BlockSpec((G, D), lambda: (0, 0)),
      out_shape=jax.ShapeDtypeStruct((G, D), jnp.float32),
  )(spatial, filt.reshape(1, D), wsp, bsp.reshape(1, D), wg1, wg2,
    bg.reshape(1, 1))


_MESH = plsc.VectorSubcoreMesh(core_axis_name="c", subcore_axis_name="s",
                               num_cores=NC, num_subcores=NS)


NB1 = EW // EB


def _alpha_body(q_hbm, k_hbm, src_hbm, dst_hbm, alpha_out, mx_out,
                srcw_v, dstw_v, qrA, krA, qrB, krB, al_v, mv_v,
                semqA, semkA, semqB, semkB):
  cid = lax.axis_index("c")
  sid = lax.axis_index("s")
  wid = sid * NC + cid
  base = wid * EW
  lane = lax.broadcasted_iota(jnp.int32, (LANES,), 0)

  pltpu.sync_copy(src_hbm.at[pl.ds(base, EW)], srcw_v)
  pltpu.sync_copy(dst_hbm.at[pl.ds(base, EW)], dstw_v)

  def fire(i, qr, kr, sq, sk):
    pltpu.async_copy(q_hbm.at[dstw_v.at[pl.ds(i * EB, EB)]], qr, sq)
    pltpu.async_copy(k_hbm.at[srcw_v.at[pl.ds(i * EB, EB)]], kr, sk)

  def drain(qr, kr, sq, sk):
    pltpu.make_async_copy(q_hbm.at[pl.ds(0, EB)], qr, sq).wait()
    pltpu.make_async_copy(k_hbm.at[pl.ds(0, EB)], kr, sk).wait()

  def compute(i, qr, kr, mvec):
    def edge_body(j, carry):
      outs = []
      for h in range(HEADS):
        acc = jnp.zeros((LANES,), jnp.float32)
        for t in range(D // LANES):
          off = h * D + t * LANES
          acc = acc + qr[j, pl.ds(off, LANES)] * kr[j, pl.ds(off, LANES)]
        sc = jnp.sum(acc) * (1.0 / 16.0)
        outs.append(jnp.where(lane == j, sc, carry[h]))
      return tuple(outs)

    avecs = lax.fori_loop(0, EB, edge_body,
                          tuple(jnp.zeros((LANES,), jnp.float32)
                                for _ in range(HEADS)))
    for h in range(HEADS):
      al_v[h, pl.ds(i * EB, EB)] = avecs[h]
    m = jnp.maximum(jnp.maximum(avecs[0], avecs[1]),
                    jnp.maximum(avecs[2], avecs[3]))
    return jnp.maximum(mvec, m)

  fire(0, qrA, krA, semqA, semkA)

  def pair_body(p, mvec):
    i0 = 2 * p
    fire(i0 + 1, qrB, krB, semqB, semkB)
    drain(qrA, krA, semqA, semkA)
    mvec = compute(i0, qrA, krA, mvec)

    @pl.when(i0 + 2 < NB1)
    def _():
      fire(i0 + 2, qrA, krA, semqA, semkA)

    drain(qrB, krB, semqB, semkB)
    return compute(i0 + 1, qrB, krB, mvec)

  mvec = lax.fori_loop(0, NB1 // 2, pair_body,
                       jnp.full((LANES,), -1e30, jnp.float32))
  mv_v[...] = mvec
  for h in range(HEADS):
    pltpu.sync_copy(al_v.at[h], alpha_out.at[h, pl.ds(base, EW)])
  pltpu.sync_copy(mv_v, mx_out.at[wid])


def sc_alpha(q, k, src, dst):
  kfn = pl.kernel(
      _alpha_body,
      out_type=(jax.ShapeDtypeStruct((HEADS, EP), jnp.float32),
                jax.ShapeDtypeStruct((NW, LANES), jnp.float32)),
      mesh=_MESH,
      scratch_types=[
          pltpu.VMEM((EW,), jnp.int32),
          pltpu.VMEM((EW,), jnp.int32),
          pltpu.VMEM((EB, HD), jnp.float32),
          pltpu.VMEM((EB, HD), jnp.float32),
          pltpu.VMEM((EB, HD), jnp.float32),
          pltpu.VMEM((EB, HD), jnp.float32),
          pltpu.VMEM((HEADS, EW), jnp.float32),
          pltpu.VMEM((LANES,), jnp.float32),
          pltpu.SemaphoreType.DMA,
          pltpu.SemaphoreType.DMA,
          pltpu.SemaphoreType.DMA,
          pltpu.SemaphoreType.DMA,
      ],
      compiler_params=pltpu.CompilerParams(needs_layout_passes=False),
  )
  return kfn(q, k, src, dst)


EPS = EP // NS
EBB = 128
NROW = EPS // EBB
KPIPE = 2
NSUPER = NROW // KPIPE
NPAIR = NSUPER // 2


def _splat(v):
  return jnp.zeros((LANES,), jnp.int32) + v


def _scatter_body(v_hbm, src2_hbm, dst2_hbm, alpha3_hbm, mx_hbm, z2_hbm,
                  z1_hbm, msg_out, den_out,
                  msh, dsh0, dsh1,
                  srb, drb, arb, mxa_v, vr0, vr1,
                  semg0, semg1, sems, semd, semi0, semi1):
  cid = lax.axis_index("c")
  sid = lax.axis_index("s")
  vrs = (vr0, vr1)
  semgs = (semg0, semg1)
  semis = (semi0, semi1)
  rbase = sid * NROW

  pltpu.sync_copy(mx_hbm, mxa_v)
  m16 = jnp.full((LANES,), -1e30, jnp.float32)
  for w in range(NW):
    m16 = jnp.maximum(m16, mxa_v[pl.ds(w * LANES, LANES)])
  gmax = jnp.max(m16)

  for ci in range(NCHUNK):
    h = ci // 2
    dsh = dsh0 if (h % 2) == 0 else dsh1
    even = (ci % 2 == 0)

    @pl.when(cid == ci // 4)
    def _(ci=ci, h=h, dsh=dsh, even=even):
      sl = pl.ds(sid * ROWS_PER_SUB, ROWS_PER_SUB)
      pltpu.sync_copy(z2_hbm.at[sl], msh.at[sl])
      if even:
        pltpu.sync_copy(z1_hbm.at[sl], dsh.at[sl])
      plsc.subcore_barrier()

      def prefetch(sb, slot):
        rows = pl.ds(rbase + sb * KPIPE, KPIPE)
        pltpu.async_copy(src2_hbm.at[rows], srb.at[slot], semis[slot])
        pltpu.async_copy(dst2_hbm.at[rows], drb.at[slot], semis[slot])
        pltpu.async_copy(alpha3_hbm.at[h, rows], arb.at[slot], semis[slot])

      def drain_prefetch(slot):
        rows = pl.ds(rbase, KPIPE)
        pltpu.make_async_copy(src2_hbm.at[rows], srb.at[slot],
                              semis[slot]).wait()
        pltpu.make_async_copy(dst2_hbm.at[rows], drb.at[slot],
                              semis[slot]).wait()
        pltpu.make_async_copy(alpha3_hbm.at[h, rows], arb.at[slot],
                              semis[slot]).wait()

      def process(slot):
        for j in range(KPIPE):
          for t in range(EBB // LANES):
            dsl = pl.ds(t * LANES, LANES)
            arb[slot, j, dsl] = jnp.exp(arb[slot, j, dsl] - gmax)
        gds = [pltpu.async_copy(v_hbm.at[ci].at[srb.at[slot, j]],
                                vrs[j], semgs[j])
               for j in range(KPIPE)]
        sds = []
        for j in range(KPIPE):
          gds[j].wait()

          def scale_t(t, _2, j=j):
            for u in range(4):
              jj = t * 4 + u
              s16 = plsc.load_gather(
                  arb, [_splat(slot), _splat(j), _splat(jj)])
              for tt in range(CW // LANES):
                vrs[j][jj, pl.ds(tt * LANES, LANES)] = (
                    vrs[j][jj, pl.ds(tt * LANES, LANES)] * s16)
            return 0

          lax.fori_loop(0, EBB // 4, scale_t, 0)
          sds.append(pltpu.async_copy(vrs[j], msh.at[drb.at[slot, j]], sems,
                                      add=True))
          if even:
            sds.append(pltpu.async_copy(arb.at[slot, j],
                                        dsh.at[drb.at[slot, j]],
                                        semd, add=True))
        for d in sds:
          d.wait()

      prefetch(0, 0)

      def pair_body(p, _):
        sb0 = 2 * p
        prefetch(sb0 + 1, 1)
        drain_prefetch(0)
        process(0)

        @pl.when(sb0 + 2 < NSUPER)
        def _():
          prefetch(sb0 + 2, 0)

        drain_prefetch(1)
        process(1)
        return 0

      lax.fori_loop(0, NPAIR, pair_body, 0)
      plsc.subcore_barrier()
      pltpu.sync_copy(msh.at[sl], msg_out.at[ci, sl])
      plsc.subcore_barrier()

  slf = pl.ds(sid * ROWS_PER_SUB, ROWS_PER_SUB)
  pltpu.sync_copy(dsh0.at[slf], den_out.at[2 * cid, slf])
  pltpu.sync_copy(dsh1.at[slf], den_out.at[2 * cid + 1, slf])


def sc_scatter(vch, src2, dst2, alpha3, mx, z2, z1):
  kfn = pl.kernel(
      _scatter_body,
      out_type=(jax.ShapeDtypeStruct((NCHUNK, NP, CW), jnp.float32),
                jax.ShapeDtypeStruct((HEADS, NP), jnp.float32)),
      mesh=_MESH,
      scratch_types=[
          pltpu.VMEM_SHARED((NP, CW), jnp.float32),
          pltpu.VMEM_SHARED((NP,), jnp.float32),
          pltpu.VMEM_SHARED((NP,), jnp.float32),
          pltpu.VMEM((2, KPIPE, EBB), jnp.int32),
          pltpu.VMEM((2, KPIPE, EBB), jnp.int32),
          pltpu.VMEM((2, KPIPE, EBB), jnp.float32),
          pltpu.VMEM((NW * LANES,), jnp.float32),
          pltpu.VMEM((EBB, CW), jnp.float32),
          pltpu.VMEM((EBB, CW), jnp.float32),
          pltpu.SemaphoreType.DMA,
          pltpu.SemaphoreType.DMA,
          pltpu.SemaphoreType.DMA,
          pltpu.SemaphoreType.DMA,
          pltpu.SemaphoreType.DMA,
          pltpu.SemaphoreType.DMA,
      ],
      compiler_params=pltpu.CompilerParams(needs_layout_passes=False),
  )
  return kfn(vch, src2, dst2, alpha3, mx, z2, z1)


def kernel(x, edge_index, batch, W0, b0, Wq, bq, Wk, bk, Wv, bv, Ws, bs,
           filt, Wsp, bsp, Wg, bg):
  src = jnp.pad(edge_index[0], (0, EP - E), constant_values=NP - 1)
  dst = jnp.pad(edge_index[1], (0, EP - E), constant_values=NP - 1)
  batch3 = jnp.pad(batch, (0, NP - N), constant_values=G).reshape(
      NP // BM, 1, BM)
  xp = jnp.pad(x, ((0, NP - N), (0, 0)))
  z2 = jnp.zeros((NP, CW), jnp.float32)
  z1 = jnp.zeros((NP,), jnp.float32)

  h = tc_matmul(xp, W0, b0)
  for l in range(NL):
    q, k, vch, s = tc_qkvs(h, Wq[l], bq[l], Wk[l], bk[l], Wv[l], bv[l],
                           Ws[l], bs[l])
    alpha4, mx = sc_alpha(q, k, src, dst)
    msg, denom = sc_scatter(vch, src.reshape(EP // EBB, EBB),
                            dst.reshape(EP // EBB, EBB),
                            alpha4.reshape(HEADS, EP // EBB, EBB),
                            mx.reshape(NW * LANES), z2, z1)
    h = tc_combine(msg, denom, s)
  spatial = tc_pool(h, batch3)
  return tc_head(spatial, filt, Wsp, bsp, Wg, bg)
